# baseline (speedup 1.0000x reference)
import jax
import jax.numpy as jnp
from jax import lax
from jax.experimental import pallas as pl
from jax.experimental.pallas import tpu as pltpu
from jax.experimental.pallas import tpu_sc as plsc

NUM_CORES = 2
NUM_SUBCORES = 16
NUM_WORKERS = NUM_CORES * NUM_SUBCORES

LATENT_DIM = 32
RPC = 4
RECW = RPC * LATENT_DIM
BATCH = 16384
ROWS_PER_WORKER = BATCH // NUM_WORKERS
WAVE = 128
WAVES = ROWS_PER_WORKER // WAVE

TC_BLK = 512


def _gather_body(emb_u4, emb_i4, cu_hbm, ci_hbm, gu_hbm, gi_hbm,
                 cu_v, ci_v, u0, u1, i0, i1,
                 sem_u, sem_i, sem_wu, sem_wi):
  wid = lax.axis_index("s") * NUM_CORES + lax.axis_index("c")
  base = pl.multiple_of(wid * ROWS_PER_WORKER, WAVE)

  pltpu.sync_copy(cu_hbm, cu_v)
  pltpu.sync_copy(ci_hbm, ci_v)

  def issue(w, ub, ib):
    s = pl.ds(pl.multiple_of(base + w * WAVE, WAVE), WAVE)
    pltpu.async_copy(emb_u4.at[cu_v.at[s]], ub, sem_u)
    pltpu.async_copy(emb_i4.at[ci_v.at[s]], ib, sem_i)

  def drain_gather(ub, ib):
    pltpu.make_async_copy(emb_u4.at[pl.ds(0, WAVE)], ub, sem_u).wait()
    pltpu.make_async_copy(emb_i4.at[pl.ds(0, WAVE)], ib, sem_i).wait()

  def writeback(w, ub, ib):
    s = pl.ds(pl.multiple_of(base + w * WAVE, WAVE), WAVE)
    pltpu.async_copy(ub, gu_hbm.at[s], sem_wu)
    pltpu.async_copy(ib, gi_hbm.at[s], sem_wi)

  def drain_writeback(ub, ib):
    pltpu.make_async_copy(u0, gu_hbm.at[pl.ds(0, WAVE)], sem_wu).wait()
    pltpu.make_async_copy(i0, gi_hbm.at[pl.ds(0, WAVE)], sem_wi).wait()

  issue(0, u0, i0)
  issue(1, u1, i1)
  drain_gather(u0, i0)
  writeback(0, u0, i0)
  drain_gather(u1, i1)
  writeback(1, u1, i1)
  drain_writeback(u0, i0)
  issue(2, u0, i0)
  drain_writeback(u1, i1)
  issue(3, u1, i1)
  drain_gather(u0, i0)
  writeback(2, u0, i0)
  drain_gather(u1, i1)
  writeback(3, u1, i1)
  drain_writeback(u0, i0)
  drain_writeback(u1, i1)


def _sc_gather(cu, ci, emb_u4, emb_i4):
  mesh = plsc.VectorSubcoreMesh(
      core_axis_name="c", subcore_axis_name="s",
      num_cores=NUM_CORES, num_subcores=NUM_SUBCORES)
  run = pl.kernel(
      _gather_body,
      out_type=(jax.ShapeDtypeStruct((BATCH, RECW), jnp.float32),
                jax.ShapeDtypeStruct((BATCH, RECW), jnp.float32)),
      mesh=mesh,
      scratch_types=[
          pltpu.VMEM((BATCH,), jnp.int32),
          pltpu.VMEM((BATCH,), jnp.int32),
          pltpu.VMEM((WAVE, RECW), jnp.float32),
          pltpu.VMEM((WAVE, RECW), jnp.float32),
          pltpu.VMEM((WAVE, RECW), jnp.float32),
          pltpu.VMEM((WAVE, RECW), jnp.float32),
          pltpu.SemaphoreType.DMA,
          pltpu.SemaphoreType.DMA,
          pltpu.SemaphoreType.DMA,
          pltpu.SemaphoreType.DMA,
      ],
  )
  return run(emb_u4, emb_i4, cu, ci)


def _dot_body(gu_ref, gi_ref, ru_ref, ri_ref, out_ref):
  gu = gu_ref[...].reshape(TC_BLK, RPC, LATENT_DIM)
  gi = gi_ref[...].reshape(TC_BLK, RPC, LATENT_DIM)
  ru = ru_ref[...]
  ri = ri_ref[...]
  u_sel = jnp.zeros((TC_BLK, LATENT_DIM), jnp.float32)
  i_sel = jnp.zeros((TC_BLK, LATENT_DIM), jnp.float32)
  for b in range(RPC):
    u_sel = u_sel + jnp.where(ru == b, gu[:, b, :], 0.0)
    i_sel = i_sel + jnp.where(ri == b, gi[:, b, :], 0.0)
  dot = jnp.sum(u_sel * i_sel, axis=1)
  out_ref[...] = 1.0 / (1.0 + jnp.exp(-dot))


def _tc_dot(gu, gi, ru, ri):
  grid = (BATCH // TC_BLK,)
  return pl.pallas_call(
      _dot_body,
      grid=grid,
      in_specs=[
          pl.BlockSpec((TC_BLK, RECW), lambda i: (i, 0)),
          pl.BlockSpec((TC_BLK, RECW), lambda i: (i, 0)),
          pl.BlockSpec((TC_BLK, 1), lambda i: (i, 0)),
          pl.BlockSpec((TC_BLK, 1), lambda i: (i, 0)),
      ],
      out_specs=pl.BlockSpec((TC_BLK,), lambda i: (i,)),
      out_shape=jax.ShapeDtypeStruct((BATCH,), jnp.float32),
  )(gu, gi, ru, ri)


@jax.jit
def _gmf(user_idx, item_idx, emb_u4, emb_i4):
  cu = user_idx // RPC
  ru = user_idx % RPC
  ci = item_idx // RPC
  ri = item_idx % RPC
  gu, gi = _sc_gather(cu, ci, emb_u4, emb_i4)
  return _tc_dot(gu, gi, ru.reshape(-1, 1), ri.reshape(-1, 1))


def kernel(user_indices, item_indices, emb_user, emb_item):
  batch = user_indices.shape[0]
  nrec = emb_user.shape[0] // RPC
  out = _gmf(user_indices.astype(jnp.int32), item_indices.astype(jnp.int32),
             emb_user.reshape(nrec, RECW), emb_item.reshape(nrec, RECW))
  return out.reshape(batch)


# speedup vs baseline: 1.0852x; 1.0852x over previous
import jax
import jax.numpy as jnp
from jax import lax
from jax.experimental import pallas as pl
from jax.experimental.pallas import tpu as pltpu
from jax.experimental.pallas import tpu_sc as plsc

NUM_CORES = 2
NUM_SUBCORES = 16
LANES = 16
NUM_WORKERS = NUM_CORES * NUM_SUBCORES

LATENT_DIM = 32
RPC = 4
RECW = RPC * LATENT_DIM
ROWS_PER_WORKER = 512
GROUPS = ROWS_PER_WORKER // LANES
SUB = 64
WAVES = ROWS_PER_WORKER // SUB
GPW = SUB // LANES


def _gmf_body(emb_u4, emb_i4, uidx_hbm, iidx_hbm, out_hbm,
              uidx_v, iidx_v, cu_v, ci_v, u_rec, i_rec, out_v,
              sem_u0, sem_i0, sem_u1, sem_i1):
  wid = lax.axis_index("s") * NUM_CORES + lax.axis_index("c")
  base = wid * ROWS_PER_WORKER

  pltpu.sync_copy(uidx_hbm.at[pl.ds(base, ROWS_PER_WORKER)], uidx_v)
  pltpu.sync_copy(iidx_hbm.at[pl.ds(base, ROWS_PER_WORKER)], iidx_v)

  for g in range(GROUPS):
    e0 = g * LANES
    cu_v[pl.ds(e0, LANES)] = uidx_v[pl.ds(e0, LANES)] // RPC
    ci_v[pl.ds(e0, LANES)] = iidx_v[pl.ds(e0, LANES)] // RPC

  def issue(w, buf, sem_u, sem_i):
    s = pl.ds(w * SUB, SUB)
    pltpu.async_copy(emb_u4.at[cu_v.at[s]], u_rec.at[buf], sem_u)
    pltpu.async_copy(emb_i4.at[ci_v.at[s]], i_rec.at[buf], sem_i)

  def drain(sem_u, sem_i):
    pltpu.make_async_copy(emb_u4.at[pl.ds(0, SUB)], u_rec.at[0], sem_u).wait()
    pltpu.make_async_copy(emb_i4.at[pl.ds(0, SUB)], i_rec.at[0], sem_i).wait()

  lane = lax.iota(jnp.int32, LANES)

  def compute(w, buf):
    bsel = jnp.full((LANES,), buf, jnp.int32)
    for gg in range(GPW):
      e0 = w * SUB + gg * LANES
      u16 = uidx_v[pl.ds(e0, LANES)]
      i16 = iidx_v[pl.ds(e0, LANES)]
      col_u = (u16 % RPC) * LATENT_DIM
      col_i = (i16 % RPC) * LATENT_DIM
      row = gg * LANES + lane
      acc = jnp.zeros((LANES,), jnp.float32)
      for d in range(LATENT_DIM):
        uv = plsc.load_gather(u_rec, [bsel, row, col_u + d])
        iv = plsc.load_gather(i_rec, [bsel, row, col_i + d])
        acc = acc + uv * iv
      rating = 1.0 / (1.0 + jnp.exp(-acc))
      out_v[pl.ds(e0, LANES)] = rating

  issue(0, 0, sem_u0, sem_i0)
  issue(1, 1, sem_u1, sem_i1)

  def pair(p, carry):
    w = 2 * p
    drain(sem_u0, sem_i0)
    compute(w, 0)

    @pl.when(p < WAVES // 2 - 1)
    def _():
      issue(w + 2, 0, sem_u0, sem_i0)

    drain(sem_u1, sem_i1)
    compute(w + 1, 1)

    @pl.when(p < WAVES // 2 - 1)
    def _():
      issue(w + 3, 1, sem_u1, sem_i1)

    return carry

  lax.fori_loop(0, WAVES // 2, pair, 0, unroll=False)

  pltpu.sync_copy(out_v, out_hbm.at[pl.ds(base, ROWS_PER_WORKER)])


@jax.jit
def _gmf(user_idx, item_idx, emb_u4, emb_i4):
  mesh = plsc.VectorSubcoreMesh(
      core_axis_name="c", subcore_axis_name="s",
      num_cores=NUM_CORES, num_subcores=NUM_SUBCORES)
  run = pl.kernel(
      _gmf_body,
      out_type=jax.ShapeDtypeStruct((NUM_WORKERS * ROWS_PER_WORKER,), jnp.float32),
      mesh=mesh,
      compiler_params=pltpu.CompilerParams(
          needs_layout_passes=False, use_tc_tiling_on_sc=False),
      scratch_types=[
          pltpu.VMEM((ROWS_PER_WORKER,), jnp.int32),
          pltpu.VMEM((ROWS_PER_WORKER,), jnp.int32),
          pltpu.VMEM((ROWS_PER_WORKER,), jnp.int32),
          pltpu.VMEM((ROWS_PER_WORKER,), jnp.int32),
          pltpu.VMEM((2, SUB, RECW), jnp.float32),
          pltpu.VMEM((2, SUB, RECW), jnp.float32),
          pltpu.VMEM((ROWS_PER_WORKER,), jnp.float32),
          pltpu.SemaphoreType.DMA,
          pltpu.SemaphoreType.DMA,
          pltpu.SemaphoreType.DMA,
          pltpu.SemaphoreType.DMA,
      ],
  )
  return run(emb_u4, emb_i4, user_idx, item_idx)


def kernel(user_indices, item_indices, emb_user, emb_item):
  batch = user_indices.shape[0]
  nrec = emb_user.shape[0] // RPC
  out = _gmf(user_indices.astype(jnp.int32), item_indices.astype(jnp.int32),
             emb_user.reshape(nrec, RECW), emb_item.reshape(nrec, RECW))
  return out.reshape(batch)
